# Initial kernel scaffold; baseline (speedup 1.0000x reference)
import jax
import jax.numpy as jnp
from jax.experimental import pallas as pl


def kernel(x, pos, edge_index, edge_attr, node_dist, recent_pos, params):
    raise NotImplementedError("write your pallas kernel here")


# speedup vs baseline: 10.9036x; 10.9036x over previous
import functools

import numpy as np
import jax
import jax.numpy as jnp
from jax import lax
from jax.experimental import pallas as pl
from jax.experimental.pallas import tpu as pltpu
from jax.experimental.pallas import tpu_sc as plsc

N = 10000
E = 160000
HID = 128
GRID = 16
RADIUS = 0.145
R2 = RADIUS * RADIUS

N_PAD = 10240
E_PAD = 163840
NW = 32
PER_W = E_PAD // NW
CH = 128
ITERS = PER_W // CH

BN = 1024
BE = 2048
BQ = 512
NP = BQ * 9

F32 = jnp.float32


def _np_dft_mats():
    TR = np.zeros((256, 144), np.float32)
    TI = np.zeros((256, 144), np.float32)
    for p in range(256):
        x = np.zeros((GRID, GRID))
        x[p // GRID, p % GRID] = 1.0
        F = np.fft.rfft2(x)
        TR[p] = F.real.reshape(-1)
        TI[p] = F.imag.reshape(-1)
    VR = np.zeros((144, 256), np.float32)
    VI = np.zeros((144, 256), np.float32)
    for f in range(144):
        Z = np.zeros((GRID, GRID // 2 + 1), complex)
        Z[f // 9, f % 9] = 1.0
        VR[f] = np.fft.irfft2(Z, s=(GRID, GRID)).reshape(-1)
        Z[f // 9, f % 9] = 1j
        VI[f] = np.fft.irfft2(Z, s=(GRID, GRID)).reshape(-1)
    return TR, TI, VR, VI


_TR, _TI, _VR, _VI = _np_dft_mats()

_m = np.linspace(-1.0, 1.0, GRID)
_g = np.stack(np.meshgrid(_m, _m, indexing='xy'))
_LQ = np.transpose(_g, (1, 2, 0)).reshape(-1, 2).astype(np.float32)


def _ln(v, g, be):
    m = jnp.mean(v, axis=-1, keepdims=True)
    va = jnp.mean((v - m) * (v - m), axis=-1, keepdims=True)
    return (v - m) * lax.rsqrt(va + 1e-5) * g + be


_gelu = jax.nn.gelu
_relu = jax.nn.relu


def _dot(a, b):
    return jnp.dot(a, b, precision=lax.Precision.HIGHEST)


def _full(shape):
    return pl.BlockSpec(shape, lambda *_: tuple(0 for _ in shape))


def _node_enc_body(x_r, pos_r,
                   embW0_r, posW0_r, b0_r, W1_r, b1_r, W2_r, b2_r, g_r, be_r,
                   Wa_r, Wb_r, Wn0a_r, bn0_r,
                   nf_o, ps_o, pd_o, qn_o):
    xb = x_r[...]
    iota = lax.broadcasted_iota(jnp.int32, (BN, 16), 1)
    oh = (iota == xb).astype(F32)
    h = _relu(_dot(oh, embW0_r[...]) + _dot(pos_r[...], posW0_r[...])
              + b0_r[...])
    h = _relu(_dot(h, W1_r[...]) + b1_r[...])
    nf = _ln(_dot(h, W2_r[...]) + b2_r[...], g_r[...], be_r[...])
    nf_o[...] = nf
    ps_o[...] = _dot(nf, Wa_r[...])
    pd_o[...] = _dot(nf, Wb_r[...])
    qn_o[...] = _dot(nf, Wn0a_r[...]) + bn0_r[...]


def _edge_enc_body(ea_r, W0_r, b0_r, W1_r, b1_r, W2_r, b2_r, g_r, be_r,
                   Wc1_r, bc1_r, Wc2_r, bc2_r, c1_o, c2_o):
    h = _relu(_dot(ea_r[...], W0_r[...]) + b0_r[...])
    h = _relu(_dot(h, W1_r[...]) + b1_r[...])
    ef = _ln(_dot(h, W2_r[...]) + b2_r[...], g_r[...], be_r[...])
    c1_o[...] = _dot(ef, Wc1_r[...]) + bc1_r[...]
    c2_o[...] = _dot(ef, Wc2_r[...]) + bc2_r[...]


def _edge_mlp1_body(s_r, d_r, c_r, W1_r, b1_r, W2_r, b2_r, g_r, be_r, Wc2_r,
                    msg_o, cp_o):
    h0 = _relu(s_r[...] + d_r[...] + c_r[...])
    h1 = _relu(_dot(h0, W1_r[...]) + b1_r[...])
    en = _ln(_dot(h1, W2_r[...]) + b2_r[...], g_r[...], be_r[...])
    msg_o[...] = en
    cp_o[...] = _dot(en, Wc2_r[...])


def _edge_mlp2_body(s_r, d_r, c2a_r, cp_r, W1_r, b1_r, W2_r, b2_r, g_r, be_r,
                    msg_o):
    h0 = _relu(s_r[...] + d_r[...] + c2a_r[...] + cp_r[...])
    h1 = _relu(_dot(h0, W1_r[...]) + b1_r[...])
    en = _ln(_dot(h1, W2_r[...]) + b2_r[...], g_r[...], be_r[...])
    msg_o[...] = en


def _node_mlp1_body(qn_r, agA_r, agB_r, nf_r, rp_r, nd_r,
                    Wn0b_r, Wn1_r, bn1_r, Wn2_r, bn2_r, gn_r, ben_r,
                    Wo0_r, bo0_r, Wo1_r, bo1_r, Wo2_r, bo2_r,
                    P2d_r, pb2_r, K0p_r, K0o_r, bk0_r, Qg0_r,
                    a1_o, ago_o, p2d_o):
    agg = (agA_r[...] + agB_r[...]) * nd_r[...][:, 0:1]
    u = _relu(qn_r[...] + _dot(agg, Wn0b_r[...]))
    u = _relu(_dot(u, Wn1_r[...]) + bn1_r[...])
    nn = _ln(_dot(u, Wn2_r[...]) + bn2_r[...], gn_r[...], ben_r[...])
    nf1 = nf_r[...] + nn
    t = _relu(_dot(nf1, Wo0_r[...]) + bo0_r[...])
    t = _relu(_dot(t, Wo1_r[...]) + bo1_r[...])
    out1 = _dot(t, Wo2_r[...]) + bo2_r[...]
    rp = rp_r[...]
    pos2d = jnp.tanh(_dot(rp, P2d_r[...]) + pb2_r[...])
    valid = rp[:, 3:4]
    pos2d = pos2d + (1.0 - valid) * 1e4
    p2 = jnp.sum(pos2d * pos2d, axis=-1, keepdims=True)
    a1_o[...] = (_dot(pos2d, K0p_r[...]) + _dot(out1, K0o_r[...])
                 + bk0_r[...])
    ago_o[...] = _dot(pos2d, Qg0_r[...])
    p2d_o[...] = jnp.concatenate(
        [pos2d, p2, jnp.zeros((BN, 5), F32)], axis=1)


def _patch_cells(p2d):
    px = p2d[:, 0:1]
    py = p2d[:, 1:2]
    jx = jnp.floor((px + 1.0) * 7.5 + 0.5)
    iy = jnp.floor((py + 1.0) * 7.5 + 0.5)
    offi = lax.broadcasted_iota(jnp.int32, (1, 9), 1)
    off = offi.astype(F32)
    dy = jnp.floor(off / 3.0)
    dx = off - dy * 3.0
    cx = jx + dx - 1.0
    cy = iy + dy - 1.0
    valid = ((cx >= 0.0) & (cx <= 15.0)
             & (cy >= 0.0) & (cy <= 15.0)).astype(F32)
    cxc = jnp.clip(cx, 0.0, 15.0)
    cyc = jnp.clip(cy, 0.0, 15.0)
    cell = (cyc * 16.0 + cxc).astype(jnp.int32)
    lqx = cxc * (2.0 / 15.0) - 1.0
    lqy = cyc * (2.0 / 15.0) - 1.0
    d2 = (px - lqx) ** 2 + (py - lqy) ** 2
    m = valid * (d2 <= R2).astype(F32)
    return cell, m, lqx, lqy


def _gno_in_body(a1_r, p2d_r, K0p_r, K1_r, k1_r, K2_r, k2_r, out_o):
    i = pl.program_id(0)
    cell, m, lqx, lqy = _patch_cells(p2d_r[...])
    b1g = (lqx[:, :, None] * K0p_r[0:1, :][None]
           + lqy[:, :, None] * K0p_r[1:2, :][None])
    h1 = _gelu(a1_r[...][:, None, :] - b1g).reshape(NP, 32)
    h2 = _gelu(_dot(h1, K1_r[...]) + k1_r[...])
    k = (_dot(h2, K2_r[...]) + k2_r[...]).reshape(BQ, 9, 8)
    rows = jnp.concatenate(
        [k * m[:, :, None], m[:, :, None], jnp.zeros((BQ, 9, 7), F32)],
        axis=2).reshape(NP, 16)
    oh = (lax.broadcasted_iota(jnp.int32, (BQ, 9, 256), 2)
          == cell[:, :, None]).astype(F32).reshape(NP, 256)
    contrib = lax.dot_general(oh, rows, (((0,), (0,)), ((), ())),
                              precision=lax.Precision.HIGHEST)

    @pl.when(i == 0)
    def _():
        out_o[...] = jnp.zeros((256, 16), F32)
    out_o[...] += contrib


def _fno_body(qsT_r, TR_r, TI_r, VR_r, VI_r,
              L1t_r, l1b_r, L2t_r, l2b_r,
              sk0_r, sk0b_r, m10_r, m10b_r, m20_r, m20b_r, wr0_r, wi0_r,
              sk1_r, sk1b_r, m11_r, m11b_r, m21_r, m21b_r, wr1_r, wi1_r,
              p1t_r, p1b_r, p2t_r, p2b_r, out_o):
    qsT = qsT_r[...]
    in_cp = qsT[0:3, :] / jnp.maximum(qsT[3:4, :], 1.0)
    h = _gelu(_dot(L1t_r[...], in_cp) + l1b_r[...])
    h = _dot(L2t_r[...], h) + l2b_r[...]
    blocks = ((sk0_r, sk0b_r, m10_r, m10b_r, m20_r, m20b_r, wr0_r, wi0_r),
              (sk1_r, sk1b_r, m11_r, m11b_r, m21_r, m21b_r, wr1_r, wi1_r))
    for (sk_r, skb_r, m1_r, m1b_r, m2_r, m2b_r, wr_r, wi_r) in blocks:
        xp = _gelu(h)
        xs = jnp.tanh(xp)
        ftR = _dot(xs, TR_r[...])
        ftI = _dot(xs, TI_r[...])
        wr = wr_r[...]
        wi = wi_r[...]
        outR = jnp.sum(ftR[:, None, :] * wr - ftI[:, None, :] * wi, axis=0)
        outI = jnp.sum(ftR[:, None, :] * wi + ftI[:, None, :] * wr, axis=0)
        spec = _dot(outR, VR_r[...]) + _dot(outI, VI_r[...])
        h = spec + _dot(sk_r[...], xp) + skb_r[...]
        y = _gelu(h)
        y = _gelu(_dot(m1_r[...], y) + m1b_r[...])
        y = _dot(m2_r[...], y) + m2b_r[...]
        h = h + y
    h = _gelu(_dot(p1t_r[...], h) + p1b_r[...])
    out_o[...] = _dot(p2t_r[...], h) + p2b_r[...]


def _gno_out_body(ago_r, p2d_r, lat_r, Q0_r, q0_r, Q1_r, q1_r, Q2_r, q2b_r,
                  pW_r, pb_r, Wa2_r, Wb2_r, Wn0a2_r, bn02_r,
                  ps_o, pd_o, qn_o, nf2_o):
    cell, m, lqx, lqy = _patch_cells(p2d_r[...])
    oh = (lax.broadcasted_iota(jnp.int32, (BQ, 9, 256), 2)
          == cell[:, :, None]).astype(F32).reshape(NP, 256)
    latg = _dot(oh, lat_r[...])
    bqg = (lqx[:, :, None] * Q0_r[0:1, :][None]
           + lqy[:, :, None] * Q0_r[1:2, :][None]
           + q0_r[...][None])
    h1 = _gelu(bqg - ago_r[...][:, None, :]).reshape(NP, 32)
    h2 = _gelu(_dot(h1, Q1_r[...]) + q1_r[...])
    k = _dot(h2, Q2_r[...]) + q2b_r[...]
    acc = jnp.sum(k.reshape(BQ, 9, 128) * latg.reshape(BQ, 9, 128)
                  * m[:, :, None], axis=1)
    cnt = jnp.sum(m, axis=1, keepdims=True)
    go = acc / jnp.maximum(cnt, 1.0)
    nf2 = _dot(go, pW_r[...]) + pb_r[...]
    ps_o[...] = _dot(nf2, Wa2_r[...])
    pd_o[...] = _dot(nf2, Wb2_r[...])
    qn_o[...] = _dot(nf2, Wn0a2_r[...]) + bn02_r[...]
    nf2_o[...] = nf2


def _node_mlp2_body(qn_r, agA_r, agB_r, nf_r, nd_r,
                    Wn0b_r, Wn1_r, bn1_r, Wn2_r, bn2_r, gn_r, ben_r,
                    Wo0_r, bo0_r, Wo1_r, bo1_r, Wo2_r, bo2_r, out_o):
    agg = (agA_r[...] + agB_r[...]) * nd_r[...][:, 0:1]
    u = _relu(qn_r[...] + _dot(agg, Wn0b_r[...]))
    u = _relu(_dot(u, Wn1_r[...]) + bn1_r[...])
    nn = _ln(_dot(u, Wn2_r[...]) + bn2_r[...], gn_r[...], ben_r[...])
    nf3 = nf_r[...] + nn
    t = _relu(_dot(nf3, Wo0_r[...]) + bo0_r[...])
    t = _relu(_dot(t, Wo1_r[...]) + bo1_r[...])
    out_o[...] = _dot(t, Wo2_r[...]) + bo2_r[...]


@functools.cache
def _sc_gather_kernel():
    mesh = plsc.VectorSubcoreMesh(core_axis_name="c", subcore_axis_name="s")

    @functools.partial(
        pl.kernel, mesh=mesh,
        out_type=[jax.ShapeDtypeStruct((E_PAD, 128), F32),
                  jax.ShapeDtypeStruct((E_PAD, 128), F32)],
        scratch_types=[pltpu.VMEM((CH,), jnp.int32),
                       pltpu.VMEM((CH,), jnp.int32),
                       pltpu.VMEM((CH, 128), F32),
                       pltpu.VMEM((CH, 128), F32),
                       pltpu.SemaphoreType.DMA,
                       pltpu.SemaphoreType.DMA])
    def gather(src_h, dst_h, tabS_h, tabD_h, outS_h, outD_h,
               idxS_v, idxD_v, bufS_v, bufD_v, semS, semD):
        wid = lax.axis_index("s") * 2 + lax.axis_index("c")
        base = wid * PER_W

        def it(i, carry):
            off = base + i * CH
            pltpu.sync_copy(src_h.at[pl.ds(off, CH)], idxS_v)
            pltpu.sync_copy(dst_h.at[pl.ds(off, CH)], idxD_v)
            cA = pltpu.async_copy(tabS_h.at[idxS_v], bufS_v, semS)
            cB = pltpu.async_copy(tabD_h.at[idxD_v], bufD_v, semD)
            cA.wait()
            cB.wait()
            pltpu.sync_copy(bufS_v, outS_h.at[pl.ds(off, CH)])
            pltpu.sync_copy(bufD_v, outD_h.at[pl.ds(off, CH)])
            return carry

        lax.fori_loop(0, ITERS, it, 0)

    return gather


@functools.cache
def _sc_scatter_kernel():
    mesh = plsc.VectorSubcoreMesh(core_axis_name="c", subcore_axis_name="s")

    @functools.partial(
        pl.kernel, mesh=mesh,
        out_type=jax.ShapeDtypeStruct((2, N_PAD, 128), F32),
        scratch_types=[pltpu.VMEM((CH,), jnp.int32),
                       pltpu.VMEM((CH, 128), F32),
                       pltpu.VMEM_SHARED((N_PAD, 128), F32)])
    def scatter(dst_h, msg_h, zero_h, out_h, idx_v, buf_v, shared):
        cid = lax.axis_index("c")
        sid = lax.axis_index("s")
        stripe = N_PAD // 16
        pltpu.sync_copy(zero_h.at[pl.ds(sid * stripe, stripe)],
                        shared.at[pl.ds(sid * stripe, stripe)])
        plsc.subcore_barrier()
        wid = sid * 2 + cid
        base = wid * PER_W

        def it(i, carry):
            off = base + i * CH
            pltpu.sync_copy(dst_h.at[pl.ds(off, CH)], idx_v)
            pltpu.sync_copy(msg_h.at[pl.ds(off, CH)], buf_v)
            pltpu.sync_copy(buf_v, shared.at[idx_v], add=True)
            return carry

        lax.fori_loop(0, ITERS, it, 0)
        plsc.subcore_barrier()
        pltpu.sync_copy(shared.at[pl.ds(sid * stripe, stripe)],
                        out_h.at[cid, pl.ds(sid * stripe, stripe)])

    return scatter


def _sc_gather(src, dst, tabS, tabD):
    return _sc_gather_kernel()(src, dst, tabS, tabD)


def _sc_scatter(dst, msg, zero):
    return _sc_scatter_kernel()(dst, msg, zero)


def kernel(x, pos, edge_index, edge_attr, node_dist, recent_pos, params):
    p = params
    f32 = F32

    def padr(a, rows, cols=None):
        padc = 0 if cols is None else cols - a.shape[1]
        return jnp.pad(a, ((0, rows - a.shape[0]), (0, padc)))

    x2 = padr(x.astype(jnp.int32)[:, None], N_PAD)
    posp = padr(pos, N_PAD, 24)
    nd8 = padr(node_dist, N_PAD, 8)
    rp8 = padr(jnp.concatenate(
        [recent_pos, jnp.ones((N, 1), f32)], axis=1), N_PAD, 8)
    eap = padr(edge_attr, E_PAD, 8)
    srcp = jnp.pad(edge_index[0].astype(jnp.int32), (0, E_PAD - E),
                   constant_values=N)
    dstp = jnp.pad(edge_index[1].astype(jnp.int32), (0, E_PAD - E),
                   constant_values=N)
    zeros_h = jnp.zeros((N_PAD, 128), f32)

    def rw(b):
        return b.reshape(1, -1)

    (nW0, nb0), (nW1, nb1), (nW2, nb2) = p['node_in']
    ng, nbe = p['node_in_ln']
    embW0 = padr(p['embed'] @ nW0[:16], 16)
    posW0 = padr(nW0[16:], 24)
    (eW0, eb0), (eW1, eb1), (eW2, eb2) = p['edge_in']
    eg, ebe = p['edge_in_ln']
    eW0p = padr(eW0, 8)
    (iE0, ib0), (iE1, ib1), (iE2, ib2) = p['in0_edge']
    ieg, iebe = p['in0_edge_ln']
    (oE0, ob0), (oE1, ob1), (oE2, ob2) = p['out0_edge']
    oeg, oebe = p['out0_edge_ln']
    (iN0, inb0), (iN1, inb1), (iN2, inb2) = p['in0_node']
    ing, inbe = p['in0_node_ln']
    (oN0, onb0), (oN1, onb1), (oN2, onb2) = p['out0_node']
    ong, onbe = p['out0_node_ln']
    (wo0, wob0), (wo1, wob1), (wo2, wob2) = p['node_out']
    wo2p = jnp.pad(wo2, ((0, 0), (0, 5)))
    wob2p = jnp.pad(wob2, (0, 5))
    P2dW, P2db = p['project2d']
    P2dp = padr(P2dW, 8)
    (K0, k0b), (K1, k1b), (K2, k2b) = p['gno_in_k']
    K2p = jnp.pad(K2, ((0, 0), (0, 5)))
    k2bp = jnp.pad(k2b, (0, 5))
    (Q0, q0b), (Q1, q1b), (Q2, q2b_) = p['gno_out_k']
    pfW, pfb = p['proj_final']

    grid_n = N_PAD // BN
    grid_e = E_PAD // BE

    def nspec(c):
        return pl.BlockSpec((BN, c), lambda i: (i, 0))

    def espec(c):
        return pl.BlockSpec((BE, c), lambda i: (i, 0))

    nf, P1s, P1d, Qnf = pl.pallas_call(
        _node_enc_body,
        grid=(grid_n,),
        in_specs=[nspec(1), nspec(24)] + [
            _full(s) for s in [(16, 128), (24, 128), (1, 128), (128, 128),
                               (1, 128), (128, 128), (1, 128), (1, 128),
                               (1, 128), (128, 128), (128, 128), (128, 128),
                               (1, 128)]],
        out_specs=[nspec(128), nspec(128), nspec(128), nspec(128)],
        out_shape=[jax.ShapeDtypeStruct((N_PAD, 128), f32),
                   jax.ShapeDtypeStruct((N_PAD, 128), f32),
                   jax.ShapeDtypeStruct((N_PAD, 128), f32),
                   jax.ShapeDtypeStruct((N_PAD, 128), f32)],
    )(x2, posp, embW0, posW0, rw(nb0), nW1, rw(nb1), nW2, rw(nb2),
      rw(ng), rw(nbe), iE0[:128], iE0[128:256], iN0[:128], rw(inb0))

    C1, C2a = pl.pallas_call(
        _edge_enc_body,
        grid=(grid_e,),
        in_specs=[espec(8)] + [
            _full(s) for s in [(8, 128), (1, 128), (128, 128), (1, 128),
                               (128, 128), (1, 128), (1, 128), (1, 128),
                               (128, 128), (1, 128), (128, 128), (1, 128)]],
        out_specs=[espec(128), espec(128)],
        out_shape=[jax.ShapeDtypeStruct((E_PAD, 128), f32),
                   jax.ShapeDtypeStruct((E_PAD, 128), f32)],
    )(eap, eW0p, rw(eb0), eW1, rw(eb1), eW2, rw(eb2), rw(eg), rw(ebe),
      iE0[256:], rw(ib0), oE0[256:], rw(ob0))

    S1, D1 = _sc_gather(srcp, dstp, P1s, P1d)
    msg1, Cp = pl.pallas_call(
        _edge_mlp1_body,
        grid=(grid_e,),
        in_specs=[espec(128), espec(128), espec(128)] + [
            _full(s) for s in [(128, 128), (1, 128), (128, 128), (1, 128),
                               (1, 128), (1, 128), (128, 128)]],
        out_specs=[espec(128), espec(128)],
        out_shape=[jax.ShapeDtypeStruct((E_PAD, 128), f32),
                   jax.ShapeDtypeStruct((E_PAD, 128), f32)],
    )(S1, D1, C1, iE1, rw(ib1), iE2, rw(ib2), rw(ieg), rw(iebe), oE0[256:])
    agg1 = _sc_scatter(dstp, msg1, zeros_h)

    A1, Ago, P2D = pl.pallas_call(
        _node_mlp1_body,
        grid=(grid_n,),
        in_specs=[nspec(128), nspec(128), nspec(128), nspec(128), nspec(8),
                  nspec(8)] + [
            _full(s) for s in [(128, 128), (128, 128), (1, 128), (128, 128),
                               (1, 128), (1, 128), (1, 128),
                               (128, 128), (1, 128), (128, 128), (1, 128),
                               (128, 8), (1, 8),
                               (8, 2), (1, 2), (2, 32), (8, 32), (1, 32),
                               (2, 32)]],
        out_specs=[nspec(32), nspec(32), nspec(8)],
        out_shape=[jax.ShapeDtypeStruct((N_PAD, 32), f32),
                   jax.ShapeDtypeStruct((N_PAD, 32), f32),
                   jax.ShapeDtypeStruct((N_PAD, 8), f32)],
    )(Qnf, agg1[0], agg1[1], nf, rp8, nd8,
      iN0[128:], iN1, rw(inb1), iN2, rw(inb2), rw(ing), rw(inbe),
      wo0, rw(wob0), wo1, rw(wob1), wo2p, rw(wob2p),
      P2dp, rw(P2db), K0[:2], jnp.pad(K0[2:], ((0, 5), (0, 0))), rw(k0b), Q0)
    def qspec(c):
        return pl.BlockSpec((BQ, c), lambda i: (i, 0))

    qsums = pl.pallas_call(
        _gno_in_body,
        grid=(N_PAD // BQ,),
        in_specs=[qspec(32), qspec(8)] + [
            _full(s) for s in [(2, 32), (32, 64), (1, 64), (64, 8), (1, 8)]],
        out_specs=_full((256, 16)),
        out_shape=jax.ShapeDtypeStruct((256, 16), f32),
    )(A1, P2D, K0[:2], K1, rw(k1b), K2p, rw(k2bp))

    (L1W, l1b), (L2W, l2b) = p['fno_lift1'], p['fno_lift2']
    (p1W, p1b), (p2W, p2b) = p['fno_proj1'], p['fno_proj2']
    fb = p['fno_blocks']

    def cb(b):
        return b.reshape(-1, 1)

    fno_args = [qsums.T, jnp.asarray(_TR), jnp.asarray(_TI),
                jnp.asarray(_VR), jnp.asarray(_VI),
                L1W.T, cb(l1b), L2W.T, cb(l2b)]
    fno_specs = [_full(s) for s in [(16, 256), (256, 144), (256, 144),
                                    (144, 256), (144, 256),
                                    (32, 3), (32, 1), (32, 32), (32, 1)]]
    for bp in fb:
        skW, skb = bp['skip']
        m1W, m1b = bp['mlp1']
        m2W, m2b = bp['mlp2']
        fno_args += [skW.T, cb(skb), m1W.T, cb(m1b), m2W.T, cb(m2b),
                     bp['wr'].reshape(32, 32, 144),
                     bp['wi'].reshape(32, 32, 144)]
        fno_specs += [_full(s) for s in [(32, 32), (32, 1), (16, 32), (16, 1),
                                         (32, 16), (32, 1), (32, 32, 144),
                                         (32, 32, 144)]]
    fno_args += [p1W.T, cb(p1b), p2W.T, cb(p2b)]
    fno_specs += [_full(s) for s in [(32, 32), (32, 1), (128, 32), (128, 1)]]
    hcp = pl.pallas_call(
        _fno_body,
        grid=(1,),
        in_specs=fno_specs,
        out_specs=_full((128, 256)),
        out_shape=jax.ShapeDtypeStruct((128, 256), f32),
    )(*fno_args)
    latent = hcp.reshape(256, 128)

    P2s, P2d, Qnf2, nf2 = pl.pallas_call(
        _gno_out_body,
        grid=(N_PAD // BQ,),
        in_specs=[qspec(32), qspec(8)] + [
            _full(s) for s in [(256, 128), (2, 32), (1, 32),
                               (32, 64), (1, 64), (64, 128), (1, 128),
                               (128, 128), (1, 128), (128, 128), (128, 128),
                               (128, 128), (1, 128)]],
        out_specs=[qspec(128), qspec(128), qspec(128), qspec(128)],
        out_shape=[jax.ShapeDtypeStruct((N_PAD, 128), f32),
                   jax.ShapeDtypeStruct((N_PAD, 128), f32),
                   jax.ShapeDtypeStruct((N_PAD, 128), f32),
                   jax.ShapeDtypeStruct((N_PAD, 128), f32)],
    )(Ago, P2D, latent, Q0, rw(q0b), Q1, rw(q1b), Q2, rw(q2b_),
      pfW, rw(pfb), oE0[:128], oE0[128:256], oN0[:128], rw(onb0))

    S2, D2 = _sc_gather(srcp, dstp, P2s, P2d)
    msg2 = pl.pallas_call(
        _edge_mlp2_body,
        grid=(grid_e,),
        in_specs=[espec(128), espec(128), espec(128), espec(128)] + [
            _full(s) for s in [(128, 128), (1, 128), (128, 128), (1, 128),
                               (1, 128), (1, 128)]],
        out_specs=espec(128),
        out_shape=jax.ShapeDtypeStruct((E_PAD, 128), f32),
    )(S2, D2, C2a, Cp, oE1, rw(ob1), oE2, rw(ob2), rw(oeg), rw(oebe))
    agg2 = _sc_scatter(dstp, msg2, zeros_h)

    res = pl.pallas_call(
        _node_mlp2_body,
        grid=(grid_n,),
        in_specs=[nspec(128), nspec(128), nspec(128), nspec(128),
                  nspec(8)] + [
            _full(s) for s in [(128, 128), (128, 128), (1, 128), (128, 128),
                               (1, 128), (1, 128), (1, 128),
                               (128, 128), (1, 128), (128, 128), (1, 128),
                               (128, 8), (1, 8)]],
        out_specs=nspec(8),
        out_shape=jax.ShapeDtypeStruct((N_PAD, 8), f32),
    )(Qnf2, agg2[0], agg2[1], nf2, nd8,
      oN0[128:], oN1, rw(onb1), oN2, rw(onb2), rw(ong), rw(onbe),
      wo0, rw(wob0), wo1, rw(wob1), wo2p, rw(wob2p))

    return res[:N, :3]
